# baseline (speedup 1.0000x reference)
import functools
import math

import jax
import jax.numpy as jnp
from jax import lax
from jax.experimental import pallas as pl
from jax.experimental.pallas import tpu as pltpu
from jax.experimental.pallas import tpu_sc as plsc

D_MODEL = 64
SCALE = math.sqrt(D_MODEL)

_NC, _NS, _LANES = 2, 16, 16
_NW = _NC * _NS
_CHUNK = 512


def _make_sc_gather(B: int):
    assert B % (_NW * _CHUNK) == 0
    b_per_w = B // _NW
    chunks = b_per_w // _CHUNK
    mesh = plsc.VectorSubcoreMesh(core_axis_name="c", subcore_axis_name="s")

    @functools.partial(
        pl.kernel,
        mesh=mesh,
        out_type=jax.ShapeDtypeStruct((B, D_MODEL), jnp.float32),
        scratch_types=[
            pltpu.VMEM((_CHUNK,), jnp.int32),
            pltpu.VMEM((_CHUNK, D_MODEL), jnp.float32),
            pltpu.SemaphoreType.DMA,
        ],
        compiler_params=pltpu.CompilerParams(use_tc_tiling_on_sc=False),
    )
    def sc_gather(x_hbm, table_hbm, out_hbm, idx_v, rows_v, sem):
        wid = lax.axis_index("s") * _NC + lax.axis_index("c")
        base = wid * b_per_w

        def chunk_body(c, carry):
            off = base + c * _CHUNK
            pltpu.sync_copy(x_hbm.at[pl.ds(off, _CHUNK)], idx_v)
            pltpu.async_copy(table_hbm.at[idx_v], rows_v, sem).wait()

            def row_body(r, rcarry):
                for j in range(D_MODEL // _LANES):
                    sl = (r, pl.ds(j * _LANES, _LANES))
                    rows_v[sl] = rows_v[sl] * SCALE
                return rcarry

            lax.fori_loop(0, _CHUNK, row_body, 0)
            pltpu.sync_copy(rows_v, out_hbm.at[pl.ds(off, _CHUNK)])
            return carry

        lax.fori_loop(0, chunks, chunk_body, 0)

    return sc_gather


_SC_GATHER = _make_sc_gather(4096 * 200)


def kernel(x, table):
    orig_shape = x.shape
    x_flat = x.reshape((-1,)).astype(jnp.int32)
    out = _SC_GATHER(x_flat, table)
    return out.reshape(orig_shape + (D_MODEL,))


# speedup vs baseline: 1.4109x; 1.4109x over previous
import functools
import math

import jax
import jax.numpy as jnp
from jax import lax
from jax.experimental import pallas as pl
from jax.experimental.pallas import tpu as pltpu
from jax.experimental.pallas import tpu_sc as plsc

D_MODEL = 64
OUT_W = 128
SCALE = math.sqrt(D_MODEL)

_NC, _NS, _LANES = 2, 16, 16
_NW = _NC * _NS
_CHUNK = 512
_NBUF = 2


def _make_sc_gather(B: int):
    assert B % (_NW * _CHUNK * _NBUF) == 0
    b_per_w = B // _NW
    chunks = b_per_w // _CHUNK
    groups = chunks // _NBUF
    mesh = plsc.VectorSubcoreMesh(core_axis_name="c", subcore_axis_name="s")

    @functools.partial(
        pl.kernel,
        mesh=mesh,
        out_type=jax.ShapeDtypeStruct((B, OUT_W), jnp.float32),
        scratch_types=[
            pltpu.VMEM((_NBUF, _CHUNK), jnp.int32),
            pltpu.VMEM((_NBUF, _CHUNK, D_MODEL), jnp.float32),
            pltpu.SemaphoreType.DMA,
            pltpu.SemaphoreType.DMA,
        ],
        compiler_params=pltpu.CompilerParams(use_tc_tiling_on_sc=False),
    )
    def sc_gather(x_hbm, table_hbm, out_hbm, idx_v, rows_v, gsem, osem):
        wid = lax.axis_index("s") * _NC + lax.axis_index("c")
        base = wid * b_per_w

        def start_gather(c, b):
            off = base + c * _CHUNK
            pltpu.sync_copy(x_hbm.at[pl.ds(off, _CHUNK)], idx_v.at[b])
            pltpu.async_copy(table_hbm.at[idx_v.at[b]], rows_v.at[b], gsem)

        def wait_gather(b):
            pltpu.make_async_copy(
                table_hbm.at[idx_v.at[b]], rows_v.at[b], gsem
            ).wait()

        def start_out(c, b):
            off = base + c * _CHUNK
            pltpu.async_copy(
                rows_v.at[b],
                out_hbm.at[pl.ds(off, _CHUNK), pl.ds(0, D_MODEL)],
                osem,
            )

        def wait_out(b):
            pltpu.make_async_copy(
                rows_v.at[b],
                out_hbm.at[pl.ds(0, _CHUNK), pl.ds(0, D_MODEL)],
                osem,
            ).wait()

        def scale(b):
            def row_body(r, carry):
                for j in range(D_MODEL // _LANES):
                    sl = (b, r, pl.ds(j * _LANES, _LANES))
                    rows_v[sl] = rows_v[sl] * SCALE
                return carry

            lax.fori_loop(0, _CHUNK, row_body, 0)

        start_gather(0, 0)

        def group_body(g, carry):
            for b in range(_NBUF):
                c = g * _NBUF + b
                o = (b + 1) % _NBUF

                @pl.when(c + 1 < chunks)
                def _():
                    @pl.when(c >= 1)
                    def _():
                        wait_out(o)

                    start_gather(c + 1, o)

                wait_gather(b)
                scale(b)
                start_out(c, b)
            return carry

        lax.fori_loop(0, groups, group_body, 0)
        for b in range(min(_NBUF, chunks)):
            wait_out(b)

    return sc_gather


_SC_GATHER = _make_sc_gather(4096 * 200)


def kernel(x, table):
    orig_shape = x.shape
    x_flat = x.reshape((-1,)).astype(jnp.int32)
    out = _SC_GATHER(x_flat, table)
    out = out.reshape(orig_shape + (OUT_W,))
    return out[..., :D_MODEL]
